# baseline (speedup 1.0000x reference)
import jax
import jax.numpy as jnp
from jax.experimental import pallas as pl
from jax.experimental.pallas import tpu as pltpu

B, W, D, H, OUT = 512, 5, 256, 128, 10

_BR = 12 * H


def _gru(h, wl, bi, bh):
    g = jax.lax.dot_general(
        h.astype(jnp.bfloat16), wl, (((1,), (1,)), ((), ())),
        preferred_element_type=jnp.float32
    ) + bi
    t = jnp.tanh(0.5 * (g[:, :4 * H] + bh[:, :4 * H]))
    r = 0.5 + 0.5 * t[:, :2 * H]
    zc = 0.5 - 0.5 * t[:, 2 * H:]
    n = jnp.tanh(g[:, 4 * H:] + r * bh[:, 4 * H:])
    return zc * n


def _brow(ref, k):
    r = _BR + 3 * k
    return jnp.concatenate(
        [ref[r:r + 1, :], ref[r + 1:r + 2, :], ref[r + 2:r + 3, :]], axis=1)


def _fused_kernel(x_ref, wb_ref, g1_ref, g2_ref, fw_ref, out_ref):
    xt = x_ref[:]
    out1 = _gru(xt, wb_ref[:6 * H, :], _brow(wb_ref, 0), _brow(wb_ref, 1))
    out2 = _gru(out1, wb_ref[6 * H:12 * H, :], _brow(wb_ref, 2),
                _brow(wb_ref, 3))
    m = jnp.sum(out2, axis=0, keepdims=True) * (1.0 / B)
    v1 = jnp.dot(m, g1_ref[:],
                 preferred_element_type=jnp.float32) + wb_ref[_BR + 12:_BR + 13, :H]
    v2 = jnp.dot(v1, g2_ref[:],
                 preferred_element_type=jnp.float32) + wb_ref[_BR + 15:_BR + 16, :64]
    o = jax.lax.dot_general(
        v2, fw_ref[:], (((1,), (1,)), ((), ())),
        preferred_element_type=jnp.float32) + wb_ref[_BR + 18:_BR + 19, :OUT]
    out_ref[:] = jnp.broadcast_to(o, (B, OUT))


def _pack_w(wf, wr):
    return jnp.concatenate(
        [wf[:H], wr[:H], wf[H:2 * H], wr[H:2 * H], wf[2 * H:], wr[2 * H:]],
        axis=0)


def _pack_b(bf, br):
    return jnp.concatenate(
        [bf[:H], br[:H], bf[H:2 * H], br[H:2 * H], bf[2 * H:], br[2 * H:]])


def _pad_row(v):
    return jnp.pad(v, (0, 6 * H - v.shape[0]))


@jax.jit
def kernel(x, g1_wih_f, g1_bih_f, g1_bhh_f, g1_wih_r, g1_bih_r, g1_bhh_r,
           g2_wih_f, g2_bih_f, g2_bhh_f, g2_wih_r, g2_bih_r, g2_bhh_r,
           gcn1_w, gcn1_b, gcn2_w, gcn2_b, fc_w, fc_b):
    xf = x.reshape(B, W * D)
    biases = jnp.concatenate(
        [_pack_b(g1_bih_f, g1_bih_r), _pack_b(g1_bhh_f, g1_bhh_r),
         _pack_b(g2_bih_f, g2_bih_r), _pack_b(g2_bhh_f, g2_bhh_r),
         _pad_row(gcn1_b), _pad_row(gcn2_b), _pad_row(fc_b)]
    ).reshape(21, 2 * H)
    wb = jnp.concatenate([_pack_w(g1_wih_f, g1_wih_r),
                          _pack_w(g2_wih_f, g2_wih_r),
                          biases], axis=0).astype(jnp.bfloat16)
    return pl.pallas_call(
        _fused_kernel,
        grid=(1,),
        out_shape=jax.ShapeDtypeStruct((B, OUT), jnp.float32),
        in_specs=[pl.BlockSpec((B, D), lambda i: (0, W - 1))]
        + [pl.BlockSpec(memory_space=pltpu.VMEM)] * 4,
        out_specs=pl.BlockSpec(memory_space=pltpu.VMEM),
    )(xf, wb, gcn1_w, gcn2_w, fc_w)


# speedup vs baseline: 1.3503x; 1.3503x over previous
import jax
import jax.numpy as jnp
from jax.experimental import pallas as pl
from jax.experimental.pallas import tpu as pltpu

B, W, D, H, OUT = 512, 5, 256, 128, 10


def _gates(g, bh):
    t = jnp.tanh(0.5 * (g[:, :2 * H] + bh[:, :2 * H]))
    r = 0.5 + 0.5 * t[:, :H]
    zc = 0.5 - 0.5 * t[:, H:]
    n = jnp.tanh(g[:, 2 * H:] + r * bh[:, 2 * H:])
    return zc * n


def _bigru(h, wl, bi, bh):
    g = jax.lax.dot_general(
        h.astype(jnp.bfloat16), wl, (((1,), (1,)), ((), ())),
        preferred_element_type=jnp.float32
    ) + bi
    return jnp.concatenate(
        [_gates(g[:, :3 * H], bh[:, :3 * H]),
         _gates(g[:, 3 * H:], bh[:, 3 * H:])], axis=1)


def _fused_kernel(x_ref, wg_ref, bb_ref, g1_ref, g2_ref, fw_ref, out_ref):
    xt = x_ref[:]
    out1 = _bigru(xt, wg_ref[:6 * H, :], bb_ref[0:1, :], bb_ref[1:2, :])
    out2 = _bigru(out1, wg_ref[6 * H:, :], bb_ref[2:3, :], bb_ref[3:4, :])
    m = jnp.sum(out2, axis=0, keepdims=True) * (1.0 / B)
    v1 = jnp.dot(m, g1_ref[:],
                 preferred_element_type=jnp.float32) + bb_ref[4:5, :H]
    v2 = jnp.dot(v1, g2_ref[:],
                 preferred_element_type=jnp.float32) + bb_ref[5:6, :64]
    o = jax.lax.dot_general(
        v2, fw_ref[:], (((1,), (1,)), ((), ())),
        preferred_element_type=jnp.float32) + bb_ref[6:7, :OUT]
    out_ref[:] = jnp.broadcast_to(o, (B, OUT))


def _pad_row(v):
    return jnp.pad(v, (0, 6 * H - v.shape[0]))


@jax.jit
def kernel(x, g1_wih_f, g1_bih_f, g1_bhh_f, g1_wih_r, g1_bih_r, g1_bhh_r,
           g2_wih_f, g2_bih_f, g2_bhh_f, g2_wih_r, g2_bih_r, g2_bhh_r,
           gcn1_w, gcn1_b, gcn2_w, gcn2_b, fc_w, fc_b):
    xf = x.reshape(B, W * D)
    cc = lambda a, b: jnp.concatenate([a, b])
    wg = jnp.concatenate(
        [g1_wih_f, g1_wih_r, g2_wih_f, g2_wih_r],
        axis=0).astype(jnp.bfloat16)
    bb = jnp.stack([cc(g1_bih_f, g1_bih_r),
                    cc(g1_bhh_f, g1_bhh_r),
                    cc(g2_bih_f, g2_bih_r),
                    cc(g2_bhh_f, g2_bhh_r),
                    _pad_row(gcn1_b),
                    _pad_row(gcn2_b),
                    _pad_row(fc_b),
                    jnp.zeros((6 * H,), jnp.float32)])
    return pl.pallas_call(
        _fused_kernel,
        grid=(1,),
        out_shape=jax.ShapeDtypeStruct((B, OUT), jnp.float32),
        in_specs=[pl.BlockSpec((B, D), lambda i: (0, W - 1))]
        + [pl.BlockSpec(memory_space=pltpu.VMEM)] * 5,
        out_specs=pl.BlockSpec(memory_space=pltpu.VMEM),
    )(xf, wg, bb, gcn1_w, gcn2_w, fc_w)


# speedup vs baseline: 1.5438x; 1.1433x over previous
import jax
import jax.numpy as jnp
from jax.experimental import pallas as pl
from jax.experimental.pallas import tpu as pltpu

B, W, D, H, OUT = 512, 5, 256, 128, 10


def _gru(h, wl, bi, bhn):
    g = jax.lax.dot_general(
        h.astype(jnp.bfloat16), wl, (((1,), (1,)), ((), ())),
        preferred_element_type=jnp.float32
    ) + bi
    t = jnp.tanh(g[:, :4 * H])
    r = 0.5 + 0.5 * t[:, :2 * H]
    zc = 0.5 - 0.5 * t[:, 2 * H:]
    n = jnp.tanh(g[:, 4 * H:] + r * bhn)
    return zc * n


def _fused_kernel(x_ref, wg_ref, bb_ref, g1_ref, g2_ref, fw_ref, out_ref):
    xt = x_ref[:]
    out1 = _gru(xt, wg_ref[:6 * H, :], bb_ref[0:1, :], bb_ref[2:3, :2 * H])
    out2 = _gru(out1, wg_ref[6 * H:, :], bb_ref[1:2, :],
                bb_ref[2:3, 2 * H:4 * H])
    m = jnp.sum(out2, axis=0, keepdims=True) * (1.0 / B)
    v1 = jnp.dot(m, g1_ref[:],
                 preferred_element_type=jnp.float32) + bb_ref[2:3, 4 * H:5 * H]
    v2 = jnp.dot(v1, g2_ref[:],
                 preferred_element_type=jnp.float32) + bb_ref[2:3,
                                                             5 * H:5 * H + 64]
    o = jax.lax.dot_general(
        v2, fw_ref[:], (((1,), (1,)), ((), ())),
        preferred_element_type=jnp.float32) + bb_ref[2:3,
                                                     5 * H + 64:5 * H + 74]
    out_ref[:] = jnp.broadcast_to(o, (B, OUT))


def _pack_w(wf, wr):
    return jnp.concatenate(
        [0.5 * wf[:H], 0.5 * wr[:H], 0.5 * wf[H:2 * H], 0.5 * wr[H:2 * H],
         wf[2 * H:], wr[2 * H:]], axis=0)


def _pack_b(bf, br):
    return jnp.concatenate(
        [bf[:H], br[:H], bf[H:2 * H], br[H:2 * H], bf[2 * H:], br[2 * H:]])


@jax.jit
def kernel(x, g1_wih_f, g1_bih_f, g1_bhh_f, g1_wih_r, g1_bih_r, g1_bhh_r,
           g2_wih_f, g2_bih_f, g2_bhh_f, g2_wih_r, g2_bih_r, g2_bhh_r,
           gcn1_w, gcn1_b, gcn2_w, gcn2_b, fc_w, fc_b):
    xf = x.reshape(B, W * D)
    wg = jnp.concatenate([_pack_w(g1_wih_f, g1_wih_r),
                          _pack_w(g2_wih_f, g2_wih_r)],
                         axis=0).astype(jnp.bfloat16)
    def combined(bi_f, bi_r, bh_f, bh_r):
        bi = _pack_b(bi_f, bi_r)
        bh = _pack_b(bh_f, bh_r)
        return jnp.concatenate(
            [0.5 * (bi[:4 * H] + bh[:4 * H]), bi[4 * H:]])
    bb = jnp.stack([
        combined(g1_bih_f, g1_bih_r, g1_bhh_f, g1_bhh_r),
        combined(g2_bih_f, g2_bih_r, g2_bhh_f, g2_bhh_r),
        jnp.concatenate(
            [g1_bhh_f[2 * H:], g1_bhh_r[2 * H:],
             g2_bhh_f[2 * H:], g2_bhh_r[2 * H:],
             gcn1_b, gcn2_b, fc_b,
             jnp.zeros((H - 64 - OUT,), jnp.float32)]),
    ])
    return pl.pallas_call(
        _fused_kernel,
        grid=(1,),
        out_shape=jax.ShapeDtypeStruct((B, OUT), jnp.float32),
        in_specs=[pl.BlockSpec((B, D), lambda i: (0, W - 1))]
        + [pl.BlockSpec(memory_space=pltpu.VMEM)] * 5,
        out_specs=pl.BlockSpec(memory_space=pltpu.VMEM),
    )(xf, wg, bb, gcn1_w, gcn2_w, fc_w)


# speedup vs baseline: 1.5512x; 1.0049x over previous
import jax
import jax.numpy as jnp
from jax.experimental import pallas as pl
from jax.experimental.pallas import tpu as pltpu

B, W, D, H, OUT = 512, 5, 256, 128, 10


def _gru(h, wl, bi, bhn):
    g = jax.lax.dot_general(
        h.astype(jnp.bfloat16), wl, (((1,), (1,)), ((), ())),
        preferred_element_type=jnp.float32
    ) + bi
    t = jnp.tanh(g[:, :4 * H])
    r = 0.5 + 0.5 * t[:, :2 * H]
    zc = 0.5 - 0.5 * t[:, 2 * H:]
    n = jnp.tanh(g[:, 4 * H:] + r * bhn)
    return zc * n


def _fused_kernel(x_ref, wg_ref, bb_ref, g1_ref, g2_ref, fw_ref, out_ref):
    xt = x_ref[:]
    out1 = _gru(xt, wg_ref[:6 * H, :], bb_ref[0:1, :], bb_ref[2:3, :2 * H])
    out2 = _gru(out1, wg_ref[6 * H:, :], bb_ref[1:2, :],
                bb_ref[2:3, 2 * H:4 * H])
    b1 = bb_ref[2:3, 4 * H:5 * H]
    b2 = bb_ref[2:3, 5 * H:5 * H + 64]
    b3 = bb_ref[2:3, 5 * H + 64:5 * H + 74]
    w12 = jnp.dot(g1_ref[:], g2_ref[:],
                  preferred_element_type=jnp.float32)
    weff = jax.lax.dot_general(
        w12, fw_ref[:], (((1,), (1,)), ((), ())),
        preferred_element_type=jnp.float32)
    b12 = jnp.dot(b1, g2_ref[:], preferred_element_type=jnp.float32) + b2
    beff = jax.lax.dot_general(
        b12, fw_ref[:], (((1,), (1,)), ((), ())),
        preferred_element_type=jnp.float32) + b3
    s = jnp.dot(jnp.full((1, B), 1.0 / B, jnp.float32), out2,
                preferred_element_type=jnp.float32)
    o = jnp.dot(s, weff, preferred_element_type=jnp.float32) + beff
    out_ref[:] = jnp.broadcast_to(o, (B, OUT))


def _pack_w(wf, wr):
    return jnp.concatenate(
        [0.5 * wf[:H], 0.5 * wr[:H], 0.5 * wf[H:2 * H], 0.5 * wr[H:2 * H],
         wf[2 * H:], wr[2 * H:]], axis=0)


def _pack_b(bf, br):
    return jnp.concatenate(
        [bf[:H], br[:H], bf[H:2 * H], br[H:2 * H], bf[2 * H:], br[2 * H:]])


@jax.jit
def kernel(x, g1_wih_f, g1_bih_f, g1_bhh_f, g1_wih_r, g1_bih_r, g1_bhh_r,
           g2_wih_f, g2_bih_f, g2_bhh_f, g2_wih_r, g2_bih_r, g2_bhh_r,
           gcn1_w, gcn1_b, gcn2_w, gcn2_b, fc_w, fc_b):
    xf = x.reshape(B, W * D)
    wg = jnp.concatenate([_pack_w(g1_wih_f, g1_wih_r),
                          _pack_w(g2_wih_f, g2_wih_r)],
                         axis=0).astype(jnp.bfloat16)
    def combined(bi_f, bi_r, bh_f, bh_r):
        bi = _pack_b(bi_f, bi_r)
        bh = _pack_b(bh_f, bh_r)
        return jnp.concatenate(
            [0.5 * (bi[:4 * H] + bh[:4 * H]), bi[4 * H:]])
    bb = jnp.stack([
        combined(g1_bih_f, g1_bih_r, g1_bhh_f, g1_bhh_r),
        combined(g2_bih_f, g2_bih_r, g2_bhh_f, g2_bhh_r),
        jnp.concatenate(
            [g1_bhh_f[2 * H:], g1_bhh_r[2 * H:],
             g2_bhh_f[2 * H:], g2_bhh_r[2 * H:],
             gcn1_b, gcn2_b, fc_b,
             jnp.zeros((H - 64 - OUT,), jnp.float32)]),
    ])
    return pl.pallas_call(
        _fused_kernel,
        grid=(1,),
        out_shape=jax.ShapeDtypeStruct((B, OUT), jnp.float32),
        in_specs=[pl.BlockSpec((B, D), lambda i: (0, W - 1))]
        + [pl.BlockSpec(memory_space=pltpu.VMEM)] * 5,
        out_specs=pl.BlockSpec(memory_space=pltpu.VMEM),
    )(xf, wg, bb, gcn1_w, gcn2_w, fc_w)
